# baseline (speedup 1.0000x reference)
import functools

import jax
import jax.numpy as jnp
from jax import lax
from jax.experimental import pallas as pl
from jax.experimental.pallas import tpu as pltpu
from jax.experimental.pallas import tpu_sc as plsc

B = 4
T = 8192
D = 1024
NC = 2
NS = 16
NW = NC * NS
PPW = T // NW
C = 8
CH = PPW // C
LANES = 16

_mesh = plsc.VectorSubcoreMesh(core_axis_name="c", subcore_axis_name="s")


@functools.partial(
    pl.kernel,
    out_type=jax.ShapeDtypeStruct((B * T, D), jnp.float32),
    mesh=_mesh,
    scratch_types=[
        pltpu.VMEM((B, CH, C), jnp.int32),
        pltpu.VMEM((2, C, D), jnp.float32),
        pltpu.VMEM((8, C, D), jnp.float32),
        pltpu.VMEM_SHARED((NS, B, C, D), jnp.float32),
        pltpu.SemaphoreType.DMA((8,)),
        pltpu.SemaphoreType.DMA((8,)),
        pltpu.SemaphoreType.DMA((B,)),
        pltpu.SemaphoreType.DMA((2,)),
    ],
)
def _embed(idx_hbm, tok_hbm, pos_hbm, out_hbm, idx_v, pos_v, tok_v, spm,
           sem_g, sem_ts, sem_w, sem_p):
    sid = lax.axis_index("s")
    wid = sid * NC + lax.axis_index("c")
    p0 = wid * PPW

    pltpu.async_copy(pos_hbm.at[pl.ds(p0, C), :], pos_v.at[0], sem_p.at[0])
    for b in range(B):
        pltpu.async_copy(idx_hbm.at[b, wid], idx_v.at[b], sem_ts.at[b])
    for b in range(B):
        pltpu.make_async_copy(idx_hbm.at[b, wid], idx_v.at[b],
                              sem_ts.at[b]).wait()
    for i in (0, 1):
        for b in range(B):
            buf = i * 4 + b
            pltpu.async_copy(tok_hbm.at[idx_v.at[b, i]], tok_v.at[buf],
                             sem_g.at[buf])

    def second_stage(pbuf, slot, row0):
        pltpu.make_async_copy(tok_v.at[pbuf], spm.at[sid, slot],
                              sem_ts.at[pbuf]).wait()
        pltpu.async_copy(spm.at[sid, slot], out_hbm.at[pl.ds(row0, C), :],
                         sem_w.at[slot])

    def drain_write(slot):
        pltpu.make_async_copy(spm.at[sid, slot], out_hbm.at[pl.ds(0, C), :],
                              sem_w.at[slot]).wait()

    def pair_body(ii, _):
        for parity in (0, 1):
            i = 2 * ii + parity
            pltpu.make_async_copy(pos_hbm.at[pl.ds(p0 + i * C, C), :],
                                  pos_v.at[parity], sem_p.at[parity]).wait()

            @pl.when(i <= CH - 2)
            def _():
                pltpu.async_copy(
                    pos_hbm.at[pl.ds(p0 + (i + 1) * C, C), :],
                    pos_v.at[1 - parity], sem_p.at[1 - parity])

            for b in range(B):
                buf = parity * 4 + b
                qbuf = (1 - parity) * 4 + b

                @pl.when(jnp.logical_and(i >= 1, i <= CH - 2))
                def _():
                    pltpu.async_copy(tok_hbm.at[idx_v.at[b, i + 1]],
                                     tok_v.at[qbuf], sem_g.at[qbuf])

                pltpu.make_async_copy(tok_hbm.at[idx_v.at[b, i]],
                                      tok_v.at[buf], sem_g.at[buf]).wait()

                @plsc.parallel_loop(0, D // LANES, unroll=4)
                def _(j):
                    sl = pl.ds(j * LANES, LANES)
                    for r in range(C):
                        plsc.addupdate(tok_v.at[buf, r, sl],
                                       pos_v[parity, r, sl])

                @pl.when(i >= 1)
                def _():
                    drain_write(b)

                pltpu.async_copy(tok_v.at[buf], spm.at[sid, b],
                                 sem_ts.at[buf])

                if b >= 1:
                    second_stage(parity * 4 + b - 1, b - 1,
                                 (b - 1) * T + p0 + i * C)
                else:
                    @pl.when(i >= 1)
                    def _():
                        second_stage((1 - parity) * 4 + 3, 3,
                                     3 * T + p0 + (i - 1) * C)
        return 0

    lax.fori_loop(0, CH // 2, pair_body, 0)

    second_stage(4 + 3, 3, 3 * T + p0 + (CH - 1) * C)
    for slot in range(B):
        drain_write(slot)


def kernel(input_ids, token_table, pos_table):
    ids = input_ids.astype(jnp.int32).reshape(B, NW, CH, C)
    out = _embed(ids, token_table, pos_table)
    return out.reshape(B, T, D)


# speedup vs baseline: 1.0028x; 1.0028x over previous
import functools

import jax
import jax.numpy as jnp
from jax import lax
from jax.experimental import pallas as pl
from jax.experimental.pallas import tpu as pltpu
from jax.experimental.pallas import tpu_sc as plsc

B = 4
T = 8192
D = 1024
NC = 2
NS = 16
NW = NC * NS
PPW = T // NW
C = 8
CH = PPW // C
LANES = 16

_mesh = plsc.VectorSubcoreMesh(core_axis_name="c", subcore_axis_name="s")


@functools.partial(
    pl.kernel,
    out_type=jax.ShapeDtypeStruct((B * T, D), jnp.float32),
    mesh=_mesh,
    scratch_types=[
        pltpu.VMEM((B, CH, C), jnp.int32),
        pltpu.VMEM((2, C, D), jnp.float32),
        pltpu.VMEM((8, C, D), jnp.float32),
        pltpu.VMEM_SHARED((NS, B, C, D), jnp.float32),
        pltpu.SemaphoreType.DMA((8,)),
        pltpu.SemaphoreType.DMA((8,)),
        pltpu.SemaphoreType.DMA((B,)),
        pltpu.SemaphoreType.DMA((2,)),
    ],
)
def _embed(idx_hbm, tok_hbm, pos_hbm, out_hbm, idx_v, pos_v, tok_v, spm,
           sem_g, sem_ts, sem_w, sem_p):
    sid = lax.axis_index("s")
    wid = sid * NC + lax.axis_index("c")
    p0 = wid * PPW

    pltpu.async_copy(pos_hbm.at[pl.ds(p0, C), :], pos_v.at[0], sem_p.at[0])
    for b in range(B):
        pltpu.async_copy(idx_hbm.at[b, wid], idx_v.at[b], sem_ts.at[b])
    for b in range(B):
        pltpu.make_async_copy(idx_hbm.at[b, wid], idx_v.at[b],
                              sem_ts.at[b]).wait()
    for i in (0, 1):
        for b in range(B):
            buf = i * 4 + b
            pltpu.async_copy(tok_hbm.at[idx_v.at[b, i]], tok_v.at[buf],
                             sem_g.at[buf])

    def second_stage(pbuf, slot, row0):
        pltpu.make_async_copy(tok_v.at[pbuf], spm.at[sid, slot],
                              sem_ts.at[pbuf]).wait()
        pltpu.async_copy(spm.at[sid, slot], out_hbm.at[pl.ds(row0, C), :],
                         sem_w.at[slot])

    def drain_write(slot):
        pltpu.make_async_copy(spm.at[sid, slot], out_hbm.at[pl.ds(0, C), :],
                              sem_w.at[slot]).wait()

    def pair_body(ii, _):
        for parity in (0, 1):
            i = 2 * ii + parity
            pltpu.make_async_copy(pos_hbm.at[pl.ds(p0 + i * C, C), :],
                                  pos_v.at[parity], sem_p.at[parity]).wait()

            @pl.when(i <= CH - 2)
            def _():
                pltpu.async_copy(
                    pos_hbm.at[pl.ds(p0 + (i + 1) * C, C), :],
                    pos_v.at[1 - parity], sem_p.at[1 - parity])

            for b in range(B):
                buf = parity * 4 + b
                qbuf = (1 - parity) * 4 + b

                @pl.when(jnp.logical_and(i >= 1, i <= CH - 2))
                def _():
                    pltpu.async_copy(tok_hbm.at[idx_v.at[b, i + 1]],
                                     tok_v.at[qbuf], sem_g.at[qbuf])

                pltpu.make_async_copy(tok_hbm.at[idx_v.at[b, i]],
                                      tok_v.at[buf], sem_g.at[buf]).wait()

                @plsc.parallel_loop(0, D // LANES, unroll=2)
                def _(j):
                    sl = pl.ds(j * LANES, LANES)
                    for r in range(C):
                        plsc.addupdate(tok_v.at[buf, r, sl],
                                       pos_v[parity, r, sl])

                @pl.when(i >= 1)
                def _():
                    drain_write(b)

                pltpu.async_copy(tok_v.at[buf], spm.at[sid, b],
                                 sem_ts.at[buf])

                if b >= 1:
                    second_stage(parity * 4 + b - 1, b - 1,
                                 (b - 1) * T + p0 + i * C)
                else:
                    @pl.when(i >= 1)
                    def _():
                        second_stage((1 - parity) * 4 + 3, 3,
                                     3 * T + p0 + (i - 1) * C)
        return 0

    lax.fori_loop(0, CH // 2, pair_body, 0)

    second_stage(4 + 3, 3, 3 * T + p0 + (CH - 1) * C)
    for slot in range(B):
        drain_write(slot)


def kernel(input_ids, token_table, pos_table):
    ids = input_ids.astype(jnp.int32).reshape(B, NW, CH, C)
    out = _embed(ids, token_table, pos_table)
    return out.reshape(B, T, D)
